# Initial kernel scaffold; baseline (speedup 1.0000x reference)
import jax
import jax.numpy as jnp
from jax.experimental import pallas as pl


def kernel(x, edge_index, edge_weights, W1, b1, W2, b2):
    raise NotImplementedError("write your pallas kernel here")


# speedup vs baseline: 10.7938x; 10.7938x over previous
import functools

import jax
import jax.numpy as jnp
from jax import lax
from jax.experimental import pallas as pl
from jax.experimental.pallas import tpu as pltpu
from jax.experimental.pallas import tpu_sc as plsc

N = 10000
D = 128
E = 320000
NPAD = 10240
E_EXT = 331776
NC = 2
NS = 16
NW = NC * NS
EPW = E_EXT // NW
C = 128
NCHUNK = EPW // C
RPS = NPAD // NS

_mesh = plsc.VectorSubcoreMesh(core_axis_name="c", subcore_axis_name="s")


def _zero_rows(buf, n_rows):
    z = jnp.zeros((16,), jnp.float32)

    def body(e, carry):
        for f in range(D // 16):
            buf[e, pl.ds(f * 16, 16)] = z
        return carry

    lax.fori_loop(0, n_rows, body, 0)


@functools.partial(
    pl.kernel,
    mesh=_mesh,
    compiler_params=pltpu.CompilerParams(needs_layout_passes=False),
    out_type=jax.ShapeDtypeStruct((NC, NPAD), jnp.float32),
    scratch_types=[
        pltpu.VMEM((C,), jnp.int32),
        pltpu.VMEM((C,), jnp.float32),
        pltpu.VMEM((RPS,), jnp.float32),
        pltpu.VMEM_SHARED((NPAD,), jnp.float32),
    ],
)
def _sc_deg(col_hbm, ew_hbm, degp_hbm, colv, ewv, zv, acc):
    cid = lax.axis_index("c")
    sid = lax.axis_index("s")
    wid = cid * NS + sid

    z = jnp.zeros((16,), jnp.float32)

    def zbody(i, carry):
        zv[pl.ds(i * 16, 16)] = z
        return carry

    lax.fori_loop(0, RPS // 16, zbody, 0)
    pltpu.sync_copy(zv, acc.at[pl.ds(sid * RPS, RPS)])
    plsc.subcore_barrier()

    def chunk(i, carry):
        base = wid * EPW + i * C
        pltpu.sync_copy(col_hbm.at[pl.ds(base, C)], colv)
        pltpu.sync_copy(ew_hbm.at[pl.ds(base, C)], ewv)
        pltpu.sync_copy(ewv, acc.at[colv], add=True)
        return carry

    lax.fori_loop(0, NCHUNK, chunk, 0)
    plsc.subcore_barrier()
    pltpu.sync_copy(acc.at[pl.ds(sid * RPS, RPS)],
                    degp_hbm.at[cid, pl.ds(sid * RPS, RPS)])


def _scale_rows(msg, normv):
    def body(j, carry):
        n16 = normv[pl.ds(j * 16, 16)]
        for e in range(16):
            s = n16[e]
            r = j * 16 + e
            for f in range(D // 16):
                sl = pl.ds(f * 16, 16)
                msg[r, sl] = msg[r, sl] * s
        return carry

    lax.fori_loop(0, C // 16, body, 0)


@functools.partial(
    pl.kernel,
    mesh=_mesh,
    compiler_params=pltpu.CompilerParams(needs_layout_passes=False),
    out_type=(jax.ShapeDtypeStruct((NC, NPAD, D), jnp.float32),
              jax.ShapeDtypeStruct((E_EXT,), jnp.float32)),
    scratch_types=[
        pltpu.VMEM((C,), jnp.int32),
        pltpu.VMEM((C,), jnp.int32),
        pltpu.VMEM((C,), jnp.float32),
        pltpu.VMEM((C,), jnp.float32),
        pltpu.VMEM((NPAD,), jnp.float32),
        pltpu.VMEM((C, D), jnp.float32),
        pltpu.VMEM_SHARED((NPAD, D), jnp.float32),
        pltpu.SemaphoreType.DMA,
    ],
)
def _sc_layer1(row_hbm, col_hbm, ew_hbm, dinv_hbm, h_hbm,
               part_hbm, norm_hbm,
               rowv, colv, ewv, normv, dinvv, msg, acc, sem):
    cid = lax.axis_index("c")
    sid = lax.axis_index("s")
    wid = cid * NS + sid

    _zero_rows(msg, C)
    for k in range(RPS // C):
        pltpu.sync_copy(msg, acc.at[pl.ds(sid * RPS + k * C, C)])
    pltpu.sync_copy(dinv_hbm, dinvv)
    plsc.subcore_barrier()

    def chunk(i, carry):
        base = wid * EPW + i * C
        pltpu.sync_copy(row_hbm.at[pl.ds(base, C)], rowv)
        pltpu.sync_copy(col_hbm.at[pl.ds(base, C)], colv)
        pltpu.sync_copy(ew_hbm.at[pl.ds(base, C)], ewv)
        gat = pltpu.async_copy(h_hbm.at[rowv], msg, sem)
        for j in range(C // 16):
            sl = pl.ds(j * 16, 16)
            dr = plsc.load_gather(dinvv, [rowv[sl]])
            dc = plsc.load_gather(dinvv, [colv[sl]])
            normv[sl] = dr * ewv[sl] * dc
        pltpu.sync_copy(normv, norm_hbm.at[pl.ds(base, C)])
        gat.wait()
        _scale_rows(msg, normv)
        pltpu.sync_copy(msg, acc.at[colv], add=True)
        return carry

    lax.fori_loop(0, NCHUNK, chunk, 0)
    plsc.subcore_barrier()
    pltpu.sync_copy(acc.at[pl.ds(sid * RPS, RPS)],
                    part_hbm.at[cid, pl.ds(sid * RPS, RPS)])


@functools.partial(
    pl.kernel,
    mesh=_mesh,
    compiler_params=pltpu.CompilerParams(needs_layout_passes=False),
    out_type=jax.ShapeDtypeStruct((NC, NPAD, D), jnp.float32),
    scratch_types=[
        pltpu.VMEM((C,), jnp.int32),
        pltpu.VMEM((C,), jnp.int32),
        pltpu.VMEM((C,), jnp.float32),
        pltpu.VMEM((C, D), jnp.float32),
        pltpu.VMEM_SHARED((NPAD, D), jnp.float32),
        pltpu.SemaphoreType.DMA,
    ],
)
def _sc_layer2(row_hbm, col_hbm, norm_hbm, h_hbm,
               part_hbm,
               rowv, colv, normv, msg, acc, sem):
    cid = lax.axis_index("c")
    sid = lax.axis_index("s")
    wid = cid * NS + sid

    _zero_rows(msg, C)
    for k in range(RPS // C):
        pltpu.sync_copy(msg, acc.at[pl.ds(sid * RPS + k * C, C)])
    plsc.subcore_barrier()

    def chunk(i, carry):
        base = wid * EPW + i * C
        pltpu.sync_copy(row_hbm.at[pl.ds(base, C)], rowv)
        pltpu.sync_copy(col_hbm.at[pl.ds(base, C)], colv)
        pltpu.sync_copy(norm_hbm.at[pl.ds(base, C)], normv)
        pltpu.async_copy(h_hbm.at[rowv], msg, sem).wait()
        _scale_rows(msg, normv)
        pltpu.sync_copy(msg, acc.at[colv], add=True)
        return carry

    lax.fori_loop(0, NCHUNK, chunk, 0)
    plsc.subcore_barrier()
    pltpu.sync_copy(acc.at[pl.ds(sid * RPS, RPS)],
                    part_hbm.at[cid, pl.ds(sid * RPS, RPS)])


_BLK = 1024
_GRID = NPAD // _BLK


def _tc_prep_body(x_ref, w1_ref, d0_ref, d1_ref, h_ref, dinv_ref):
    dinv_ref[...] = lax.rsqrt(d0_ref[...] + d1_ref[...])
    h_ref[...] = lax.dot_general(x_ref[...], w1_ref[...],
                                 (((1,), (1,)), ((), ())),
                                 preferred_element_type=jnp.float32)


def _tc_prep(x_pad, w1, d0, d1):
    return pl.pallas_call(
        _tc_prep_body,
        grid=(_GRID,),
        in_specs=[
            pl.BlockSpec((_BLK, D), lambda i: (i, 0)),
            pl.BlockSpec((D, D), lambda i: (0, 0)),
            pl.BlockSpec((8, 128), lambda i: (i, 0)),
            pl.BlockSpec((8, 128), lambda i: (i, 0)),
        ],
        out_specs=[
            pl.BlockSpec((_BLK, D), lambda i: (i, 0)),
            pl.BlockSpec((8, 128), lambda i: (i, 0)),
        ],
        out_shape=[
            jax.ShapeDtypeStruct((NPAD, D), jnp.float32),
            jax.ShapeDtypeStruct((NPAD // 128, 128), jnp.float32),
        ],
    )(x_pad, w1, d0, d1)


def _tc_mid_body(p0_ref, p1_ref, b_ref, w2_ref, h2_ref):
    a1 = jax.nn.relu(p0_ref[...] + p1_ref[...] + b_ref[...])
    h2_ref[...] = lax.dot_general(a1, w2_ref[...],
                                  (((1,), (1,)), ((), ())),
                                  preferred_element_type=jnp.float32)


def _tc_mid(p0, p1, b1, w2):
    return pl.pallas_call(
        _tc_mid_body,
        grid=(_GRID,),
        in_specs=[
            pl.BlockSpec((_BLK, D), lambda i: (i, 0)),
            pl.BlockSpec((_BLK, D), lambda i: (i, 0)),
            pl.BlockSpec((1, D), lambda i: (0, 0)),
            pl.BlockSpec((D, D), lambda i: (0, 0)),
        ],
        out_specs=pl.BlockSpec((_BLK, D), lambda i: (i, 0)),
        out_shape=jax.ShapeDtypeStruct((NPAD, D), jnp.float32),
    )(p0, p1, b1.reshape(1, D), w2)


def _tc_final_body(p0_ref, p1_ref, b_ref, out_ref):
    out_ref[...] = jax.nn.relu(p0_ref[...] + p1_ref[...] + b_ref[...])


def _tc_final(p0, p1, b2):
    return pl.pallas_call(
        _tc_final_body,
        grid=(_GRID,),
        in_specs=[
            pl.BlockSpec((_BLK, D), lambda i: (i, 0)),
            pl.BlockSpec((_BLK, D), lambda i: (i, 0)),
            pl.BlockSpec((1, D), lambda i: (0, 0)),
        ],
        out_specs=pl.BlockSpec((_BLK, D), lambda i: (i, 0)),
        out_shape=jax.ShapeDtypeStruct((NPAD, D), jnp.float32),
    )(p0, p1, b2.reshape(1, D))


def kernel(x, edge_index, edge_weights, W1, b1, W2, b2):
    row = edge_index[0]
    col = edge_index[1]
    sl = jnp.arange(NPAD, dtype=jnp.int32)
    npad_e = E_EXT - E - NPAD
    pad_i = jnp.full((npad_e,), NPAD - 1, jnp.int32)
    row_ext = jnp.concatenate([row, sl, pad_i])
    col_ext = jnp.concatenate([col, sl, pad_i])
    ew_ext = jnp.concatenate([edge_weights,
                              jnp.ones((NPAD,), jnp.float32),
                              jnp.zeros((npad_e,), jnp.float32)])
    x_pad = jnp.pad(x, ((0, NPAD - N), (0, 0)))

    degp = _sc_deg(col_ext, ew_ext)
    d0 = degp[0].reshape(NPAD // 128, 128)
    d1 = degp[1].reshape(NPAD // 128, 128)
    h1, dinv2 = _tc_prep(x_pad, W1, d0, d1)
    dinv = dinv2.reshape(NPAD)
    part1, norm_ext = _sc_layer1(row_ext, col_ext, ew_ext, dinv, h1)
    h2 = _tc_mid(part1[0], part1[1], b1, W2)
    part2 = _sc_layer2(row_ext, col_ext, norm_ext, h2)
    out = _tc_final(part2[0], part2[1], b2)
    return out[:N]
